# baseline (speedup 1.0000x reference)
import functools

import jax
import jax.numpy as jnp
from jax import lax
from jax.experimental import pallas as pl
from jax.experimental.pallas import tpu as pltpu
from jax.experimental.pallas import tpu_sc as plsc

G, S, D, E, TOP_K = 2, 2048, 2048, 8, 2
CAP = 256
C_OUT = CAP - 1
BS = 256
NS = S // BS

NSLICE = G * C_OUT * (S // 128)
NF = NSLICE * 8 * 128
ND = NSLICE * 2 * 128
NFP = NF + 8
NDP = ND + 8
NW = 32
NREC = G * S * TOP_K
RROWS = NREC // 128
SL_PER_W = NSLICE // NW
ZCH = SL_PER_W * 8 * 128 // 16


def _router_body(ec_ref, x_ref, w_ref, b_ref, vals_ref, cidx_ref, didx_ref,
                 dval_ref, carry_ref):
    g = pl.program_id(0)
    sblk = pl.program_id(1)

    @pl.when(sblk == 0)
    def _():
        carry_ref[...] = jnp.zeros_like(carry_ref)

    xb = x_ref[0]
    wb = w_ref[...]
    bb = b_ref[...]

    logits = jnp.dot(xb, wb, preferred_element_type=jnp.float32) + bb
    m = jnp.max(logits, axis=1, keepdims=True)
    u = jnp.exp(logits - m)
    q = u / jnp.sum(u, axis=1, keepdims=True)
    qi = lax.bitcast_convert_type(q, jnp.int32)
    p = lax.bitcast_convert_type(qi & jnp.int32(-65536), jnp.float32)

    iota_e = lax.broadcasted_iota(jnp.int32, (BS, E), 1)
    v1 = jnp.max(p, axis=1, keepdims=True)
    sel1 = ((p - v1) == 0).astype(jnp.int32)
    idx1 = jnp.min(iota_e * sel1 + E * (1 - sel1), axis=1, keepdims=True)
    neg_inf = jnp.array(-jnp.inf, dtype=p.dtype)
    is1 = (iota_e - idx1) == 0
    p2 = jnp.where(is1, neg_inf, p)
    v2 = jnp.max(p2, axis=1, keepdims=True)
    sel2 = ((p2 - v2) == 0).astype(jnp.int32)
    idx2 = jnp.min(iota_e * sel2 + E * (1 - sel2), axis=1, keepdims=True)
    is2 = (iota_e - idx2) == 0

    oh1 = is1.astype(jnp.float32)
    oh2 = is2.astype(jnp.float32)
    mask16 = jnp.concatenate([oh1, oh2], axis=1)
    r_i = lax.broadcasted_iota(jnp.int32, (BS, BS), 0)
    c_i = lax.broadcasted_iota(jnp.int32, (BS, BS), 1)
    tri = (c_i <= r_i).astype(jnp.float32)
    pos = jnp.dot(tri, mask16, preferred_element_type=jnp.float32)
    pos = pos + carry_ref[...]
    carry_ref[...] = pos[BS - 1 : BS, :]

    pos1 = jnp.sum(pos[:, :E] * oh1, axis=1, keepdims=True).astype(jnp.int32)
    pos2 = jnp.sum(pos[:, E:] * oh2, axis=1, keepdims=True).astype(jnp.int32)

    ec = ec_ref[0, 0]
    lim = jnp.minimum(jnp.int32(CAP), ec)
    ok1 = (pos1 <= C_OUT) & (pos1 < lim)
    ok2 = (pos2 <= C_OUT) & (pos2 < lim)

    i_tok = lax.broadcasted_iota(jnp.int32, (BS, 1), 0)
    st = sblk * (BS // 128) + (i_tok >> 7)
    sl = i_tok & 127
    sid1 = (g * C_OUT + pos1 - 1) * (S // 128) + st
    sid2 = (g * C_OUT + pos2 - 1) * (S // 128) + st
    c1 = jnp.where(ok1, sid1 * 1024 + idx1 * 128 + sl, NF)
    c2 = jnp.where(ok2, sid2 * 1024 + idx2 * 128 + sl, NF)
    d1 = jnp.where(ok1, sid1 * 256 + (idx1 >> 2) * 128 + sl, ND)
    d2 = jnp.where(ok2, sid2 * 256 + (idx2 >> 2) * 128 + sl, ND)
    b1 = jnp.int32(1) << (8 * (idx1 & 3))
    b2 = jnp.int32(1) << (8 * (idx2 & 3))
    samew = ok1 & ok2 & (d1 == d2)
    bm = jnp.where(samew, b1 | b2, 0)
    b1 = jnp.where(samew, bm, b1)
    b2 = jnp.where(samew, bm, b2)

    def two_rows(col):
        return jnp.transpose(col, (1, 0)).reshape(2, 128)

    vals_ref[0] = jnp.concatenate([two_rows(v1), two_rows(v2)], axis=0)
    cidx_ref[0] = jnp.concatenate([two_rows(c1), two_rows(c2)], axis=0)
    didx_ref[0] = jnp.concatenate([two_rows(d1), two_rows(d2)], axis=0)
    dval_ref[0] = jnp.concatenate([two_rows(b1), two_rows(b2)], axis=0)


def _router_records(x, expert_capacity, W, b):
    xb = x.astype(jnp.bfloat16)
    wb = W.astype(jnp.bfloat16)
    bb = b.astype(jnp.bfloat16).astype(jnp.float32).reshape(1, E)
    ec = jnp.asarray(expert_capacity, jnp.int32).reshape(1, 1)
    grid = (G, NS)
    rec_spec = pl.BlockSpec((1, 4, 128), lambda g, s: (g * NS + s, 0, 0))
    return pl.pallas_call(
        _router_body,
        grid=grid,
        in_specs=[
            pl.BlockSpec(memory_space=pltpu.SMEM),
            pl.BlockSpec((1, BS, D), lambda g, s: (g, s, 0)),
            pl.BlockSpec((D, E), lambda g, s: (0, 0)),
            pl.BlockSpec((1, E), lambda g, s: (0, 0)),
        ],
        out_specs=[rec_spec] * 4,
        out_shape=[
            jax.ShapeDtypeStruct((RROWS // 4, 4, 128), jnp.float32),
            jax.ShapeDtypeStruct((RROWS // 4, 4, 128), jnp.int32),
            jax.ShapeDtypeStruct((RROWS // 4, 4, 128), jnp.int32),
            jax.ShapeDtypeStruct((RROWS // 4, 4, 128), jnp.int32),
        ],
        scratch_shapes=[pltpu.VMEM((1, 2 * E), jnp.float32)],
    )(ec, xb, wb, bb)


def _sc_body(vals_hbm, cidx_hbm, didx_hbm, dval_hbm, comb_hbm, dispw_hbm,
             zf, zi, rv, rc, rd, rw, semz, sems):
    wid = lax.axis_index("s") * 2 + lax.axis_index("c")

    zeros16f = jnp.zeros((16,), jnp.float32)
    zeros16i = jnp.zeros((16,), jnp.int32)

    def memset_body(i, _):
        zf[pl.ds(i * 16, 16)] = zeros16f
        zi[pl.ds(i * 16, 16)] = zeros16i
        return _

    lax.fori_loop(0, ZCH // 16, memset_body, 0, unroll=4)

    comb_base = wid * (SL_PER_W * 8 * 128)
    disp_base = wid * (SL_PER_W * 2 * 128)

    def zfire(i, _):
        pltpu.make_async_copy(
            zf, comb_hbm.at[pl.ds(comb_base + i * ZCH, ZCH)], semz).start()
        return _

    def dfire(i, _):
        pltpu.make_async_copy(
            zi, dispw_hbm.at[pl.ds(disp_base + i * ZCH, ZCH)], semz).start()
        return _

    lax.fori_loop(0, 16, zfire, 0)
    lax.fori_loop(0, 4, dfire, 0)

    pltpu.sync_copy(vals_hbm, rv)
    pltpu.sync_copy(cidx_hbm, rc)
    pltpu.sync_copy(didx_hbm, rd)
    pltpu.sync_copy(dval_hbm, rw)

    lo = wid * SL_PER_W
    hi = lo + SL_PER_W

    def mask_body(j, _):
        off = j * 16
        cv = rc[pl.ds(off, 16)]
        sid = cv >> 10
        keep = (sid >= lo) & (sid < hi)
        rc[pl.ds(off, 16)] = jnp.where(keep, cv, NF)
        dv = rd[pl.ds(off, 16)]
        rd[pl.ds(off, 16)] = jnp.where(keep, dv, ND)
        return _

    lax.fori_loop(0, NREC // 16, mask_body, 0, unroll=4)

    def zdrain(i, _):
        pltpu.make_async_copy(
            comb_hbm.at[pl.ds(0, ZCH)], zf, semz).wait()
        return _

    lax.fori_loop(0, 20, zdrain, 0)

    h1 = pltpu.make_async_copy(rv, comb_hbm.at[rc], sems)
    h1.start()
    h2 = pltpu.make_async_copy(rw, dispw_hbm.at[rd], sems)
    h2.start()
    h1.wait()
    h2.wait()


@functools.partial(jax.jit, static_argnames=())
def kernel(x, expert_capacity, W, b):
    vals, cidx, didx, dval = (a.reshape(NREC) for a in
                              _router_records(x, expert_capacity, W, b))

    mesh = plsc.VectorSubcoreMesh(core_axis_name="c", subcore_axis_name="s")
    comb_flat, dispw = pl.kernel(
        _sc_body,
        out_type=[
            jax.ShapeDtypeStruct((NFP,), jnp.float32),
            jax.ShapeDtypeStruct((NDP,), jnp.int32),
        ],
        mesh=mesh,
        scratch_types=[
            pltpu.VMEM((ZCH,), jnp.float32),
            pltpu.VMEM((ZCH,), jnp.int32),
            pltpu.VMEM((NREC,), jnp.float32),
            pltpu.VMEM((NREC,), jnp.int32),
            pltpu.VMEM((NREC,), jnp.int32),
            pltpu.VMEM((NREC,), jnp.int32),
            pltpu.SemaphoreType.DMA,
            pltpu.SemaphoreType.DMA,
        ],
    )(vals, cidx, didx, dval)

    comb = (comb_flat[:NF]
            .reshape(G, C_OUT, S // 128, E, 128)
            .transpose(0, 2, 4, 3, 1)
            .reshape(G, S, E, C_OUT))
    d8 = lax.bitcast_convert_type(dispw[:ND], jnp.int8)
    disp = (d8.reshape(G, C_OUT, S // 128, 2, 128, 4)
            .transpose(0, 2, 4, 3, 5, 1)
            .reshape(G, S, E, C_OUT)
            .astype(jnp.bool_))
    return (comb, disp, 0.0)


# speedup vs baseline: 786.1624x; 786.1624x over previous
import functools

import jax
import jax.numpy as jnp
from jax import lax
from jax.experimental import pallas as pl
from jax.experimental.pallas import tpu as pltpu

G, S, D, E, TOP_K = 2, 2048, 2048, 8, 2
CAP = 256
C_OUT = CAP - 1
BS = 256
NS = S // BS


def _router_body(ec_ref, x_ref, w_ref, b_ref, j_ref, comb_ref, disp_ref,
                 carry_ref):
    sblk = pl.program_id(1)

    @pl.when(sblk == 0)
    def _():
        carry_ref[...] = jnp.zeros_like(carry_ref)

    xb = x_ref[0]
    wb = w_ref[...]
    bb = b_ref[...]

    logits = jnp.dot(xb, wb, preferred_element_type=jnp.float32) + bb
    m = jnp.max(logits, axis=1, keepdims=True)
    u = jnp.exp(logits - m)
    q = u / jnp.sum(u, axis=1, keepdims=True)
    qi = lax.bitcast_convert_type(q, jnp.int32)
    p = lax.bitcast_convert_type(qi & jnp.int32(-65536), jnp.float32)

    iota_e = lax.broadcasted_iota(jnp.int32, (BS, E), 1)
    v1 = jnp.max(p, axis=1, keepdims=True)
    sel1 = ((p - v1) == 0).astype(jnp.int32)
    idx1 = jnp.min(iota_e * sel1 + E * (1 - sel1), axis=1, keepdims=True)
    neg_inf = jnp.array(-jnp.inf, dtype=p.dtype)
    is1 = (iota_e - idx1) == 0
    p2 = jnp.where(is1, neg_inf, p)
    v2 = jnp.max(p2, axis=1, keepdims=True)
    sel2 = ((p2 - v2) == 0).astype(jnp.int32)
    idx2 = jnp.min(iota_e * sel2 + E * (1 - sel2), axis=1, keepdims=True)
    is2 = (iota_e - idx2) == 0

    oh1 = is1.astype(jnp.float32)
    oh2 = is2.astype(jnp.float32)
    mask16 = jnp.concatenate([oh1, oh2], axis=1)
    r_i = lax.broadcasted_iota(jnp.int32, (BS, BS), 0)
    c_i = lax.broadcasted_iota(jnp.int32, (BS, BS), 1)
    tri = (c_i <= r_i).astype(jnp.float32)
    pos = jnp.dot(tri, mask16, preferred_element_type=jnp.float32)
    pos = pos + carry_ref[...]
    carry_ref[...] = pos[BS - 1 : BS, :]

    pos1 = jnp.sum(pos[:, :E] * oh1, axis=1, keepdims=True).astype(jnp.int32)
    pos2 = jnp.sum(pos[:, E:] * oh2, axis=1, keepdims=True).astype(jnp.int32)

    ec = ec_ref[0, 0]
    lim = jnp.minimum(jnp.int32(CAP), ec)
    ok1 = (pos1 <= C_OUT) & (pos1 < lim)
    ok2 = (pos2 <= C_OUT) & (pos2 < lim)
    t1 = jnp.where(ok1, idx1 * C_OUT + pos1 - 1, -1)
    t2 = jnp.where(ok2, idx2 * C_OUT + pos2 - 1, -1)

    t1r = jnp.transpose(t1, (1, 0)).reshape(1, 1, BS)
    t2r = jnp.transpose(t2, (1, 0)).reshape(1, 1, BS)
    g1r = jnp.transpose(v1, (1, 0)).reshape(1, 1, BS)
    g2r = jnp.transpose(v2, (1, 0)).reshape(1, 1, BS)
    j3 = j_ref[...].reshape(C_OUT, E, 1)
    cmp1 = (j3 - t1r) == 0
    cmp2 = (j3 - t2r) == 0
    zero = jnp.zeros((), jnp.float32)
    comb_ref[0] = jnp.where(cmp1, g1r, jnp.where(cmp2, g2r, zero))
    disp_ref[0] = (cmp1 | cmp2).astype(jnp.int8)


@functools.partial(jax.jit, static_argnames=())
def kernel(x, expert_capacity, W, b):
    xb = x.astype(jnp.bfloat16)
    wb = W.astype(jnp.bfloat16)
    bb = b.astype(jnp.bfloat16).astype(jnp.float32).reshape(1, E)
    ec = jnp.asarray(expert_capacity, jnp.int32).reshape(1, 1)
    jarr = (jnp.arange(E, dtype=jnp.int32)[None, :] * C_OUT
            + jnp.arange(C_OUT, dtype=jnp.int32)[:, None])

    grid = (G, NS)
    comb, disp = pl.pallas_call(
        _router_body,
        grid=grid,
        in_specs=[
            pl.BlockSpec(memory_space=pltpu.SMEM),
            pl.BlockSpec((1, BS, D), lambda g, s: (g, s, 0)),
            pl.BlockSpec((D, E), lambda g, s: (0, 0)),
            pl.BlockSpec((1, E), lambda g, s: (0, 0)),
            pl.BlockSpec((C_OUT, E), lambda g, s: (0, 0)),
        ],
        out_specs=[
            pl.BlockSpec((1, C_OUT, E, BS), lambda g, s: (g, 0, 0, s)),
            pl.BlockSpec((1, C_OUT, E, BS), lambda g, s: (g, 0, 0, s)),
        ],
        out_shape=[
            jax.ShapeDtypeStruct((G, C_OUT, E, S), jnp.float32),
            jax.ShapeDtypeStruct((G, C_OUT, E, S), jnp.int8),
        ],
        scratch_shapes=[pltpu.VMEM((1, 2 * E), jnp.float32)],
    )(ec, xb, wb, bb, jarr)
    return (jnp.transpose(comb, (0, 3, 2, 1)),
            jnp.transpose(disp, (0, 3, 2, 1)).astype(jnp.bool_), 0.0)


# speedup vs baseline: 831.9339x; 1.0582x over previous
import functools

import jax
import jax.numpy as jnp
from jax import lax
from jax.experimental import pallas as pl
from jax.experimental.pallas import tpu as pltpu

G, S, D, E, TOP_K = 2, 2048, 2048, 8, 2
CAP = 256
C_OUT = CAP - 1
BS = 512
NS = S // BS


def _router_body(ec_ref, x_ref, w_ref, b_ref, j_ref, comb_ref, disp_ref,
                 carry_ref):
    sblk = pl.program_id(1)

    @pl.when(sblk == 0)
    def _():
        carry_ref[...] = jnp.zeros_like(carry_ref)

    xb = x_ref[0]
    wb = w_ref[...]
    bb = b_ref[...]

    logits = jnp.dot(xb, wb, preferred_element_type=jnp.float32) + bb
    m = jnp.max(logits, axis=1, keepdims=True)
    u = jnp.exp(logits - m)
    q = u / jnp.sum(u, axis=1, keepdims=True)
    qi = lax.bitcast_convert_type(q, jnp.int32)
    p = lax.bitcast_convert_type(qi & jnp.int32(-65536), jnp.float32)

    iota_e = lax.broadcasted_iota(jnp.int32, (BS, E), 1)
    v1 = jnp.max(p, axis=1, keepdims=True)
    sel1 = ((p - v1) == 0).astype(jnp.int32)
    idx1 = jnp.min(iota_e * sel1 + E * (1 - sel1), axis=1, keepdims=True)
    neg_inf = jnp.array(-jnp.inf, dtype=p.dtype)
    is1 = (iota_e - idx1) == 0
    p2 = jnp.where(is1, neg_inf, p)
    v2 = jnp.max(p2, axis=1, keepdims=True)
    sel2 = ((p2 - v2) == 0).astype(jnp.int32)
    idx2 = jnp.min(iota_e * sel2 + E * (1 - sel2), axis=1, keepdims=True)
    is2 = (iota_e - idx2) == 0

    oh1 = is1.astype(jnp.float32)
    oh2 = is2.astype(jnp.float32)
    mask16 = jnp.concatenate([oh1, oh2], axis=1)
    r_i = lax.broadcasted_iota(jnp.int32, (BS, BS), 0)
    c_i = lax.broadcasted_iota(jnp.int32, (BS, BS), 1)
    tri = (c_i <= r_i).astype(jnp.float32)
    pos = jnp.dot(tri, mask16, preferred_element_type=jnp.float32)
    pos = pos + carry_ref[...]
    carry_ref[...] = pos[BS - 1 : BS, :]

    pos1 = jnp.sum(pos[:, :E] * oh1, axis=1, keepdims=True).astype(jnp.int32)
    pos2 = jnp.sum(pos[:, E:] * oh2, axis=1, keepdims=True).astype(jnp.int32)

    ec = ec_ref[0, 0]
    lim = jnp.minimum(jnp.int32(CAP), ec)
    ok1 = (pos1 <= C_OUT) & (pos1 < lim)
    ok2 = (pos2 <= C_OUT) & (pos2 < lim)
    t1 = jnp.where(ok1, idx1 * C_OUT + pos1 - 1, -1)
    t2 = jnp.where(ok2, idx2 * C_OUT + pos2 - 1, -1)

    t1r = jnp.transpose(t1, (1, 0)).reshape(1, 1, BS)
    t2r = jnp.transpose(t2, (1, 0)).reshape(1, 1, BS)
    g1r = jnp.transpose(v1, (1, 0)).reshape(1, 1, BS)
    g2r = jnp.transpose(v2, (1, 0)).reshape(1, 1, BS)
    j3 = j_ref[...].reshape(C_OUT, E, 1)
    cmp1 = (j3 - t1r) == 0
    cmp2 = (j3 - t2r) == 0
    zero = jnp.zeros((), jnp.float32)
    comb_ref[0] = jnp.where(cmp1, g1r, jnp.where(cmp2, g2r, zero))
    disp_ref[0] = (cmp1 | cmp2).astype(jnp.int8)


@functools.partial(jax.jit, static_argnames=())
def kernel(x, expert_capacity, W, b):
    xb = x.astype(jnp.bfloat16)
    wb = W.astype(jnp.bfloat16)
    bb = b.astype(jnp.bfloat16).astype(jnp.float32).reshape(1, E)
    ec = jnp.asarray(expert_capacity, jnp.int32).reshape(1, 1)
    jarr = (jnp.arange(E, dtype=jnp.int32)[None, :] * C_OUT
            + jnp.arange(C_OUT, dtype=jnp.int32)[:, None])

    grid = (G, NS)
    comb, disp = pl.pallas_call(
        _router_body,
        grid=grid,
        in_specs=[
            pl.BlockSpec(memory_space=pltpu.SMEM),
            pl.BlockSpec((1, BS, D), lambda g, s: (g, s, 0)),
            pl.BlockSpec((D, E), lambda g, s: (0, 0)),
            pl.BlockSpec((1, E), lambda g, s: (0, 0)),
            pl.BlockSpec((C_OUT, E), lambda g, s: (0, 0)),
        ],
        out_specs=[
            pl.BlockSpec((1, C_OUT, E, BS), lambda g, s: (g, 0, 0, s)),
            pl.BlockSpec((1, C_OUT, E, BS), lambda g, s: (g, 0, 0, s)),
        ],
        out_shape=[
            jax.ShapeDtypeStruct((G, C_OUT, E, S), jnp.float32),
            jax.ShapeDtypeStruct((G, C_OUT, E, S), jnp.int8),
        ],
        scratch_shapes=[pltpu.VMEM((1, 2 * E), jnp.float32)],
    )(ec, xb, wb, bb, jarr)
    return (jnp.transpose(comb, (0, 3, 2, 1)),
            jnp.transpose(disp, (0, 3, 2, 1)).astype(jnp.bool_), 0.0)


# speedup vs baseline: 837.0973x; 1.0062x over previous
import functools

import jax
import jax.numpy as jnp
from jax import lax
from jax.experimental import pallas as pl
from jax.experimental.pallas import tpu as pltpu

G, S, D, E, TOP_K = 2, 2048, 2048, 8, 2
CAP = 256
C_OUT = CAP - 1
BS = 1024
NS = S // BS


def _router_body(ec_ref, x_ref, w_ref, b_ref, j_ref, comb_ref, disp_ref,
                 carry_ref):
    sblk = pl.program_id(1)

    @pl.when(sblk == 0)
    def _():
        carry_ref[...] = jnp.zeros_like(carry_ref)

    xb = x_ref[0]
    wb = w_ref[...]
    bb = b_ref[...]

    logits = jnp.dot(xb, wb, preferred_element_type=jnp.float32) + bb
    m = jnp.max(logits, axis=1, keepdims=True)
    u = jnp.exp(logits - m)
    q = u / jnp.sum(u, axis=1, keepdims=True)
    qi = lax.bitcast_convert_type(q, jnp.int32)
    p = lax.bitcast_convert_type(qi & jnp.int32(-65536), jnp.float32)

    iota_e = lax.broadcasted_iota(jnp.int32, (BS, E), 1)
    v1 = jnp.max(p, axis=1, keepdims=True)
    sel1 = ((p - v1) == 0).astype(jnp.int32)
    idx1 = jnp.min(iota_e * sel1 + E * (1 - sel1), axis=1, keepdims=True)
    neg_inf = jnp.array(-jnp.inf, dtype=p.dtype)
    is1 = (iota_e - idx1) == 0
    p2 = jnp.where(is1, neg_inf, p)
    v2 = jnp.max(p2, axis=1, keepdims=True)
    sel2 = ((p2 - v2) == 0).astype(jnp.int32)
    idx2 = jnp.min(iota_e * sel2 + E * (1 - sel2), axis=1, keepdims=True)
    is2 = (iota_e - idx2) == 0

    oh1 = is1.astype(jnp.float32)
    oh2 = is2.astype(jnp.float32)
    mask16 = jnp.concatenate([oh1, oh2], axis=1)
    r_i = lax.broadcasted_iota(jnp.int32, (BS, BS), 0)
    c_i = lax.broadcasted_iota(jnp.int32, (BS, BS), 1)
    tri = (c_i <= r_i).astype(jnp.float32)
    pos = jnp.dot(tri, mask16, preferred_element_type=jnp.float32)
    pos = pos + carry_ref[...]
    carry_ref[...] = pos[BS - 1 : BS, :]

    pos1 = jnp.sum(pos[:, :E] * oh1, axis=1, keepdims=True).astype(jnp.int32)
    pos2 = jnp.sum(pos[:, E:] * oh2, axis=1, keepdims=True).astype(jnp.int32)

    ec = ec_ref[0, 0]
    lim = jnp.minimum(jnp.int32(CAP), ec)
    ok1 = (pos1 <= C_OUT) & (pos1 < lim)
    ok2 = (pos2 <= C_OUT) & (pos2 < lim)
    t1 = jnp.where(ok1, idx1 * C_OUT + pos1 - 1, -1)
    t2 = jnp.where(ok2, idx2 * C_OUT + pos2 - 1, -1)

    t1r = jnp.transpose(t1, (1, 0)).reshape(1, 1, BS)
    t2r = jnp.transpose(t2, (1, 0)).reshape(1, 1, BS)
    g1r = jnp.transpose(v1, (1, 0)).reshape(1, 1, BS)
    g2r = jnp.transpose(v2, (1, 0)).reshape(1, 1, BS)
    j3 = j_ref[...].reshape(C_OUT, E, 1)
    cmp1 = (j3 - t1r) == 0
    cmp2 = (j3 - t2r) == 0
    zero = jnp.zeros((), jnp.float32)
    comb_ref[0] = jnp.where(cmp1, g1r, jnp.where(cmp2, g2r, zero))
    disp_ref[0] = (cmp1 | cmp2).astype(jnp.int8)


@functools.partial(jax.jit, static_argnames=())
def kernel(x, expert_capacity, W, b):
    xb = x.astype(jnp.bfloat16)
    wb = W.astype(jnp.bfloat16)
    bb = b.astype(jnp.bfloat16).astype(jnp.float32).reshape(1, E)
    ec = jnp.asarray(expert_capacity, jnp.int32).reshape(1, 1)
    jarr = (jnp.arange(E, dtype=jnp.int32)[None, :] * C_OUT
            + jnp.arange(C_OUT, dtype=jnp.int32)[:, None])

    grid = (G, NS)
    comb, disp = pl.pallas_call(
        _router_body,
        grid=grid,
        in_specs=[
            pl.BlockSpec(memory_space=pltpu.SMEM),
            pl.BlockSpec((1, BS, D), lambda g, s: (g, s, 0)),
            pl.BlockSpec((D, E), lambda g, s: (0, 0)),
            pl.BlockSpec((1, E), lambda g, s: (0, 0)),
            pl.BlockSpec((C_OUT, E), lambda g, s: (0, 0)),
        ],
        out_specs=[
            pl.BlockSpec((1, C_OUT, E, BS), lambda g, s: (g, 0, 0, s)),
            pl.BlockSpec((1, C_OUT, E, BS), lambda g, s: (g, 0, 0, s)),
        ],
        out_shape=[
            jax.ShapeDtypeStruct((G, C_OUT, E, S), jnp.float32),
            jax.ShapeDtypeStruct((G, C_OUT, E, S), jnp.int8),
        ],
        scratch_shapes=[pltpu.VMEM((1, 2 * E), jnp.float32)],
    )(ec, xb, wb, bb, jarr)
    return (jnp.transpose(comb, (0, 3, 2, 1)),
            jnp.transpose(disp, (0, 3, 2, 1)).astype(jnp.bool_), 0.0)


# speedup vs baseline: 999.6023x; 1.1941x over previous
import functools

import jax
import jax.numpy as jnp
from jax import lax
from jax.experimental import pallas as pl
from jax.experimental.pallas import tpu as pltpu

G, S, D, E, TOP_K = 2, 2048, 2048, 8, 2
CAP = 256
C_OUT = CAP - 1
BS = 1024
NS = S // BS


def _router_body(ec_ref, x_ref, w_ref, b_ref, j_ref, comb_ref, disp_ref,
                 carry_ref):
    sblk = pl.program_id(1)

    @pl.when(sblk == 0)
    def _():
        carry_ref[...] = jnp.zeros_like(carry_ref)

    xb = x_ref[0].astype(jnp.bfloat16)
    wb = w_ref[...]
    bb = b_ref[...]

    logits = jnp.dot(xb, wb, preferred_element_type=jnp.float32) + bb
    m = jnp.max(logits, axis=1, keepdims=True)
    u = jnp.exp(logits - m)
    q = u / jnp.sum(u, axis=1, keepdims=True)
    qi = lax.bitcast_convert_type(q, jnp.int32)
    p = lax.bitcast_convert_type(qi & jnp.int32(-65536), jnp.float32)

    iota_e = lax.broadcasted_iota(jnp.int32, (BS, E), 1)
    v1 = jnp.max(p, axis=1, keepdims=True)
    sel1 = ((p - v1) == 0).astype(jnp.int32)
    idx1 = jnp.min(iota_e * sel1 + E * (1 - sel1), axis=1, keepdims=True)
    neg_inf = jnp.array(-jnp.inf, dtype=p.dtype)
    is1 = (iota_e - idx1) == 0
    p2 = jnp.where(is1, neg_inf, p)
    v2 = jnp.max(p2, axis=1, keepdims=True)
    sel2 = ((p2 - v2) == 0).astype(jnp.int32)
    idx2 = jnp.min(iota_e * sel2 + E * (1 - sel2), axis=1, keepdims=True)
    is2 = (iota_e - idx2) == 0

    oh1 = is1.astype(jnp.float32)
    oh2 = is2.astype(jnp.float32)
    mask16 = jnp.concatenate([oh1, oh2], axis=1)
    r_i = lax.broadcasted_iota(jnp.int32, (BS, BS), 0)
    c_i = lax.broadcasted_iota(jnp.int32, (BS, BS), 1)
    tri = (c_i <= r_i).astype(jnp.float32)
    pos = jnp.dot(tri, mask16, preferred_element_type=jnp.float32)
    pos = pos + carry_ref[...]
    carry_ref[...] = pos[BS - 1 : BS, :]

    pos1 = jnp.sum(pos[:, :E] * oh1, axis=1, keepdims=True).astype(jnp.int32)
    pos2 = jnp.sum(pos[:, E:] * oh2, axis=1, keepdims=True).astype(jnp.int32)

    ec = ec_ref[0, 0]
    lim = jnp.minimum(jnp.int32(CAP), ec)
    ok1 = (pos1 <= C_OUT) & (pos1 < lim)
    ok2 = (pos2 <= C_OUT) & (pos2 < lim)
    t1 = jnp.where(ok1, idx1 * C_OUT + pos1 - 1, -1)
    t2 = jnp.where(ok2, idx2 * C_OUT + pos2 - 1, -1)

    t1r = jnp.transpose(t1, (1, 0)).reshape(1, 1, BS)
    t2r = jnp.transpose(t2, (1, 0)).reshape(1, 1, BS)
    g1r = jnp.transpose(v1, (1, 0)).reshape(1, 1, BS)
    g2r = jnp.transpose(v2, (1, 0)).reshape(1, 1, BS)
    j3 = j_ref[...].reshape(C_OUT, E, 1)
    cmp1 = (j3 - t1r) == 0
    cmp2 = (j3 - t2r) == 0
    zero = jnp.zeros((), jnp.float32)
    comb_ref[0] = jnp.where(cmp1, g1r, jnp.where(cmp2, g2r, zero))
    disp_ref[0] = (cmp1 | cmp2).astype(jnp.int8)


@functools.partial(jax.jit, static_argnames=())
def kernel(x, expert_capacity, W, b):
    wb = W.astype(jnp.bfloat16)
    bb = b.astype(jnp.bfloat16).astype(jnp.float32).reshape(1, E)
    ec = jnp.asarray(expert_capacity, jnp.int32).reshape(1, 1)
    jarr = (jnp.arange(E, dtype=jnp.int32)[None, :] * C_OUT
            + jnp.arange(C_OUT, dtype=jnp.int32)[:, None])

    grid = (G, NS)
    comb, disp = pl.pallas_call(
        _router_body,
        grid=grid,
        in_specs=[
            pl.BlockSpec(memory_space=pltpu.SMEM),
            pl.BlockSpec((1, BS, D), lambda g, s: (g, s, 0)),
            pl.BlockSpec((D, E), lambda g, s: (0, 0)),
            pl.BlockSpec((1, E), lambda g, s: (0, 0)),
            pl.BlockSpec((C_OUT, E), lambda g, s: (0, 0)),
        ],
        out_specs=[
            pl.BlockSpec((1, C_OUT, E, BS), lambda g, s: (g, 0, 0, s)),
            pl.BlockSpec((1, C_OUT, E, BS), lambda g, s: (g, 0, 0, s)),
        ],
        out_shape=[
            jax.ShapeDtypeStruct((G, C_OUT, E, S), jnp.float32),
            jax.ShapeDtypeStruct((G, C_OUT, E, S), jnp.int8),
        ],
        scratch_shapes=[pltpu.VMEM((1, 2 * E), jnp.float32)],
    )(ec, x, wb, bb, jarr)
    return (jnp.transpose(comb, (0, 3, 2, 1)),
            jnp.transpose(disp, (0, 3, 2, 1)).astype(jnp.bool_), 0.0)


# speedup vs baseline: 1003.5145x; 1.0039x over previous
import functools

import jax
import jax.numpy as jnp
from jax import lax
from jax.experimental import pallas as pl
from jax.experimental.pallas import tpu as pltpu

G, S, D, E, TOP_K = 2, 2048, 2048, 8, 2
CAP = 256
C_OUT = CAP - 1
BS = 512
NS = S // BS


def _router_body(ec_ref, x_ref, w_ref, b_ref, j_ref, comb_ref, disp_ref,
                 carry_ref):
    sblk = pl.program_id(1)

    @pl.when(sblk == 0)
    def _():
        carry_ref[...] = jnp.zeros_like(carry_ref)

    xb = x_ref[0].astype(jnp.bfloat16)
    wb = w_ref[...]
    bb = b_ref[...]

    logits = jnp.dot(xb, wb, preferred_element_type=jnp.float32) + bb
    m = jnp.max(logits, axis=1, keepdims=True)
    u = jnp.exp(logits - m)
    q = u / jnp.sum(u, axis=1, keepdims=True)
    qi = lax.bitcast_convert_type(q, jnp.int32)
    p = lax.bitcast_convert_type(qi & jnp.int32(-65536), jnp.float32)

    iota_e = lax.broadcasted_iota(jnp.int32, (BS, E), 1)
    v1 = jnp.max(p, axis=1, keepdims=True)
    sel1 = ((p - v1) == 0).astype(jnp.int32)
    idx1 = jnp.min(iota_e * sel1 + E * (1 - sel1), axis=1, keepdims=True)
    neg_inf = jnp.array(-jnp.inf, dtype=p.dtype)
    is1 = (iota_e - idx1) == 0
    p2 = jnp.where(is1, neg_inf, p)
    v2 = jnp.max(p2, axis=1, keepdims=True)
    sel2 = ((p2 - v2) == 0).astype(jnp.int32)
    idx2 = jnp.min(iota_e * sel2 + E * (1 - sel2), axis=1, keepdims=True)
    is2 = (iota_e - idx2) == 0

    oh1 = is1.astype(jnp.float32)
    oh2 = is2.astype(jnp.float32)
    mask16 = jnp.concatenate([oh1, oh2], axis=1)
    r_i = lax.broadcasted_iota(jnp.int32, (BS, BS), 0)
    c_i = lax.broadcasted_iota(jnp.int32, (BS, BS), 1)
    tri = (c_i <= r_i).astype(jnp.float32)
    pos = jnp.dot(tri, mask16, preferred_element_type=jnp.float32)
    pos = pos + carry_ref[...]
    carry_ref[...] = pos[BS - 1 : BS, :]

    pos1 = jnp.sum(pos[:, :E] * oh1, axis=1, keepdims=True).astype(jnp.int32)
    pos2 = jnp.sum(pos[:, E:] * oh2, axis=1, keepdims=True).astype(jnp.int32)

    ec = ec_ref[0, 0]
    lim = jnp.minimum(jnp.int32(CAP), ec)
    ok1 = (pos1 <= C_OUT) & (pos1 < lim)
    ok2 = (pos2 <= C_OUT) & (pos2 < lim)
    t1 = jnp.where(ok1, idx1 * C_OUT + pos1 - 1, -1)
    t2 = jnp.where(ok2, idx2 * C_OUT + pos2 - 1, -1)

    t1r = jnp.transpose(t1, (1, 0)).reshape(1, 1, BS)
    t2r = jnp.transpose(t2, (1, 0)).reshape(1, 1, BS)
    g1r = jnp.transpose(v1, (1, 0)).reshape(1, 1, BS)
    g2r = jnp.transpose(v2, (1, 0)).reshape(1, 1, BS)
    j3 = j_ref[...].reshape(C_OUT, E, 1)
    cmp1 = (j3 - t1r) == 0
    cmp2 = (j3 - t2r) == 0
    zero = jnp.zeros((), jnp.float32)
    comb_ref[0] = jnp.where(cmp1, g1r, jnp.where(cmp2, g2r, zero))
    disp_ref[0] = (cmp1 | cmp2).astype(jnp.int8)


@functools.partial(jax.jit, static_argnames=())
def kernel(x, expert_capacity, W, b):
    wb = W.astype(jnp.bfloat16)
    bb = b.astype(jnp.bfloat16).astype(jnp.float32).reshape(1, E)
    ec = jnp.asarray(expert_capacity, jnp.int32).reshape(1, 1)
    jarr = (jnp.arange(E, dtype=jnp.int32)[None, :] * C_OUT
            + jnp.arange(C_OUT, dtype=jnp.int32)[:, None])

    grid = (G, NS)
    comb, disp = pl.pallas_call(
        _router_body,
        grid=grid,
        in_specs=[
            pl.BlockSpec(memory_space=pltpu.SMEM),
            pl.BlockSpec((1, BS, D), lambda g, s: (g, s, 0)),
            pl.BlockSpec((D, E), lambda g, s: (0, 0)),
            pl.BlockSpec((1, E), lambda g, s: (0, 0)),
            pl.BlockSpec((C_OUT, E), lambda g, s: (0, 0)),
        ],
        out_specs=[
            pl.BlockSpec((1, C_OUT, E, BS), lambda g, s: (g, 0, 0, s)),
            pl.BlockSpec((1, C_OUT, E, BS), lambda g, s: (g, 0, 0, s)),
        ],
        out_shape=[
            jax.ShapeDtypeStruct((G, C_OUT, E, S), jnp.float32),
            jax.ShapeDtypeStruct((G, C_OUT, E, S), jnp.int8),
        ],
        scratch_shapes=[pltpu.VMEM((1, 2 * E), jnp.float32)],
    )(ec, x, wb, bb, jarr)
    return (jnp.transpose(comb, (0, 3, 2, 1)),
            jnp.transpose(disp, (0, 3, 2, 1)).astype(jnp.bool_), 0.0)
